# baseline (speedup 1.0000x reference)
import functools

import jax
import jax.numpy as jnp
from jax import lax
from jax.experimental import pallas as pl
from jax.experimental.pallas import tpu as pltpu
from jax.experimental.pallas import tpu_sc as plsc

_PAD_ID = 0
_UNK_ID = 1
_OFFSET = 32000
_EPS = 1e-20

_B, _T, _V = 2, 2048, 32104
_R = _B * _T
_L = 16
_NC, _NS = 2, 16
_NW = _NC * _NS
_RPW = _R // _NW
_VQ = _V // 8
_TQ = _T // 128

_LN2 = 0.6931471805599453
_SQRT2 = 1.4142135623730951


def _neg_ln(x):
    bits = plsc.bitcast(x, jnp.int32)
    e = (bits >> 23) - 127
    m = plsc.bitcast((bits & 0x007FFFFF) | 0x3F800000, jnp.float32)
    big = m > _SQRT2
    m = jnp.where(big, m * 0.5, m)
    e = jnp.where(big, e + 1, e)
    s = (m - 1.0) / (m + 1.0)
    z = s * s
    p = 2.0 * s * (1.0 + z * (1.0 / 3.0 + z * (0.2 + z * (1.0 / 7.0 + z / 9.0))))
    return -(e.astype(jnp.float32) * _LN2 + p)


def _sc_loss_partials(probf, al, tg):
    mesh = plsc.VectorSubcoreMesh(core_axis_name="c", subcore_axis_name="s")

    @functools.partial(
        pl.kernel,
        out_type=jax.ShapeDtypeStruct((_NC * _L,), jnp.float32),
        mesh=mesh,
        scratch_types=[
            pltpu.VMEM((_RPW,), jnp.int32),
            pltpu.VMEM((_RPW,), jnp.int32),
            pltpu.VMEM((_RPW,), jnp.int32),
            pltpu.VMEM((_RPW,), jnp.int32),
            pltpu.VMEM((_RPW,), jnp.float32),
            pltpu.VMEM((_RPW,), jnp.float32),
            pltpu.VMEM((_L,), jnp.float32),
            pltpu.VMEM_SHARED((_NS * _L,), jnp.float32),
            pltpu.VMEM((_NS * _L,), jnp.float32),
            pltpu.SemaphoreType.DMA,
        ],
        compiler_params=pltpu.CompilerParams(needs_layout_passes=False),
    )
    def k(probf_hbm, al_hbm, tg_hbm, out_hbm,
          al_v, tg_v, idx1_v, idx2_v, val1_v, val2_v, acc_v, shr_v, red_v,
          sem):
        sid = lax.axis_index("s")
        cid = lax.axis_index("c")
        wid = sid * _NC + cid
        b = wid // (_NW // _B)
        tq = wid % (_NW // _B)
        base = wid * _RPW
        pltpu.sync_copy(al_hbm.at[pl.ds(base, _RPW)], al_v)
        pltpu.sync_copy(tg_hbm.at[pl.ds(base, _RPW)], tg_v)

        tile0 = (b * _VQ * _TQ + tq) * 1024
        for j in range(_RPW // _L):
            sl = pl.ds(j * _L, _L)
            col = j * _L + lax.iota(jnp.int32, _L)
            ve = al_v[sl] + _OFFSET
            vo = tg_v[sl]
            idx1_v[sl] = tile0 + (ve >> 3) * (_TQ * 1024) + (ve & 7) * 128 + col
            idx2_v[sl] = tile0 + (vo >> 3) * (_TQ * 1024) + (vo & 7) * 128 + col

        c1 = pltpu.async_copy(probf_hbm.at[idx1_v], val1_v, sem)
        c2 = pltpu.async_copy(probf_hbm.at[idx2_v], val2_v, sem)
        c1.wait()
        c2.wait()

        acc = jnp.zeros((_L,), jnp.float32)
        for j in range(_RPW // _L):
            sl = pl.ds(j * _L, _L)
            alc = al_v[sl]
            tgc = tg_v[sl]
            g1 = val1_v[sl]
            g2 = val2_v[sl]
            al_unk = alc == _UNK_ID
            tg_unk = tgc == _UNK_ID
            extra = jnp.where(al_unk, 0.0, g1) + _EPS
            fp = extra + jnp.where(tg_unk, 0.0, g2)
            fp = fp + jnp.where(al_unk & tg_unk, g2, 0.0)
            acc = acc + _neg_ln(jnp.where(tgc == _PAD_ID, 1.0, fp))

        acc_v[...] = acc
        pltpu.sync_copy(acc_v, shr_v.at[pl.ds(sid * _L, _L)])
        plsc.subcore_barrier()

        @pl.when(sid == 0)
        def _():
            pltpu.sync_copy(shr_v, red_v)
            tot = red_v[pl.ds(0, _L)]
            for i in range(1, _NS):
                tot = tot + red_v[pl.ds(i * _L, _L)]
            acc_v[...] = jnp.broadcast_to(jnp.sum(tot), (_L,))
            pltpu.sync_copy(acc_v, out_hbm.at[pl.ds(cid * _L, _L)])

    return k(probf, al, tg)


def kernel(prob, alignment, target):
    probf = prob.reshape(_B, _TQ, 128, _VQ, 8).transpose(0, 3, 1, 4, 2)
    probf = probf.reshape(_B * _V * _T)
    part = _sc_loss_partials(probf, alignment.reshape(-1), target.reshape(-1))
    return part[0] + part[_L]


# speedup vs baseline: 1.0877x; 1.0877x over previous
import functools

import jax
import jax.numpy as jnp
from jax import lax
from jax.experimental import pallas as pl
from jax.experimental.pallas import tpu as pltpu
from jax.experimental.pallas import tpu_sc as plsc

_PAD_ID = 0
_UNK_ID = 1
_OFFSET = 32000
_EPS = 1e-20

_B, _T, _V = 2, 2048, 32104
_R = _B * _T
_L = 16
_NC, _NS = 2, 16
_NW = _NC * _NS
_RPW = _R // _NW
_VQ = _V // 8
_TQ = _T // 128


def _sc_final_prob(probf, al, tg):
    mesh = plsc.VectorSubcoreMesh(core_axis_name="c", subcore_axis_name="s")

    @functools.partial(
        pl.kernel,
        out_type=jax.ShapeDtypeStruct((_R,), jnp.float32),
        mesh=mesh,
        scratch_types=[
            pltpu.VMEM((_RPW,), jnp.int32),
            pltpu.VMEM((_RPW,), jnp.int32),
            pltpu.VMEM((_RPW,), jnp.int32),
            pltpu.VMEM((_RPW,), jnp.int32),
            pltpu.VMEM((_RPW,), jnp.float32),
            pltpu.VMEM((_RPW,), jnp.float32),
            pltpu.VMEM((_RPW,), jnp.float32),
            pltpu.SemaphoreType.DMA,
        ],
    )
    def k(probf_hbm, al_hbm, tg_hbm, out_hbm,
          al_v, tg_v, idx1_v, idx2_v, val1_v, val2_v, out_v, sem):
        wid = lax.axis_index("s") * _NC + lax.axis_index("c")
        b = wid // (_NW // _B)
        tq = wid % (_NW // _B)
        base = wid * _RPW
        pltpu.sync_copy(al_hbm.at[tq, b, :], al_v)
        pltpu.sync_copy(tg_hbm.at[tq, b, :], tg_v)

        tile0 = (b * _VQ * _TQ + tq) * 1024
        for j in range(_RPW // _L):
            sl = pl.ds(j * _L, _L)
            col = j * _L + lax.iota(jnp.int32, _L)
            ve = al_v[sl] + _OFFSET
            vo = tg_v[sl]
            idx1_v[sl] = tile0 + (ve >> 3) * (_TQ * 1024) + (ve & 7) * 128 + col
            idx2_v[sl] = tile0 + (vo >> 3) * (_TQ * 1024) + (vo & 7) * 128 + col

        c1 = pltpu.async_copy(probf_hbm.at[idx1_v], val1_v, sem)
        c2 = pltpu.async_copy(probf_hbm.at[idx2_v], val2_v, sem)
        c1.wait()
        c2.wait()

        for j in range(_RPW // _L):
            sl = pl.ds(j * _L, _L)
            alc = al_v[sl]
            tgc = tg_v[sl]
            g1 = val1_v[sl]
            g2 = val2_v[sl]
            al_unk = alc == _UNK_ID
            tg_unk = tgc == _UNK_ID
            extra = jnp.where(al_unk, 0.0, g1) + _EPS
            fp = extra + jnp.where(tg_unk, 0.0, g2)
            fp = fp + jnp.where(al_unk & tg_unk, g2, 0.0)
            out_v[sl] = jnp.where(tgc == _PAD_ID, 1.0, fp)

        pltpu.sync_copy(out_v, out_hbm.at[pl.ds(base, _RPW)])

    return k(probf, al, tg)


def _tc_neg_log_sum(fp):
    def body(fp_ref, out_ref):
        out_ref[0, 0] = -jnp.sum(jnp.log(fp_ref[...]))

    out = pl.pallas_call(
        body,
        out_shape=jax.ShapeDtypeStruct((1, 1), jnp.float32),
        in_specs=[pl.BlockSpec(memory_space=pltpu.VMEM)],
        out_specs=pl.BlockSpec(memory_space=pltpu.SMEM),
    )(fp.reshape(_R // 128, 128))
    return out[0, 0]


def kernel(prob, alignment, target):
    probf = prob.reshape(_B, _TQ, 128, _VQ, 8).transpose(0, 3, 1, 4, 2)
    probf = probf.reshape(_B * _V * _T)
    al3 = alignment.reshape(_B, _TQ, 128).transpose(1, 0, 2)
    tg3 = target.reshape(_B, _TQ, 128).transpose(1, 0, 2)
    fp = _sc_final_prob(probf, al3, tg3)
    return _tc_neg_log_sum(fp)
